# baseline (speedup 1.0000x reference)
import dataclasses
import functools

import jax
import jax.numpy as jnp
from jax import lax
from jax.experimental import pallas as pl
from jax.experimental.pallas import tpu as pltpu
from jax.experimental.pallas import tpu_sc as plsc

NC = 2
NS = 16
LANES = 16
CHUNK = 80
ZROWS = 128


def _sc_aggregate(x, src, dst, zrow):
    N, D = x.shape
    E = src.shape[0]
    NW = NC * NS
    assert E % (NW * CHUNK) == 0, (E, NW, CHUNK)
    epw = E // NW
    nchunk = epw // CHUNK
    na = nchunk // 2 + (nchunk // 2) % 2
    nb = nchunk - na
    assert na % 2 == 0 and nb % 2 == 1
    half = na * CHUNK
    bufsz = max(na, nb) * CHUNK
    rps = -(-N // (NS * ZROWS)) * ZROWS
    Np = NS * rps

    mesh = plsc.VectorSubcoreMesh(core_axis_name="c", subcore_axis_name="s")
    cp = pltpu.CompilerParams()
    if "needs_layout_passes" in pltpu.CompilerParams.__dataclass_fields__:
        cp = dataclasses.replace(cp, needs_layout_passes=False)

    @functools.partial(
        pl.kernel,
        out_type=[
            jax.ShapeDtypeStruct((NC * Np, D), jnp.float32),
            jax.ShapeDtypeStruct((NW * Np,), jnp.float32),
        ],
        mesh=mesh,
        compiler_params=cp,
        scratch_types=[
            pltpu.VMEM_SHARED((Np, D), jnp.float32),
            pltpu.VMEM((bufsz,), jnp.int32),
            pltpu.VMEM((epw,), jnp.int32),
            pltpu.VMEM((CHUNK, D), jnp.float32),
            pltpu.VMEM((CHUNK, D), jnp.float32),
            pltpu.VMEM((Np,), jnp.float32),
            pltpu.SemaphoreType.DMA,
            pltpu.SemaphoreType.DMA,
            pltpu.SemaphoreType.DMA,
            pltpu.SemaphoreType.DMA,
        ],
    )
    def agg_kernel(x_hbm, src_hbm, dst_hbm, zrow_hbm, agg_hbm, deg_hbm,
                   acc_sh, src_half, dst_all, rows0, rows1, deg_loc,
                   sem0, sem1, ssem0, ssem1):
        c = lax.axis_index("c")
        s = lax.axis_index("s")
        w = c * NS + s
        base_r = s * rps
        ebase = w * epw
        cp_z = pltpu.async_copy(zrow_hbm.at[pl.ds(base_r, rps)],
                                acc_sh.at[pl.ds(base_r, rps)], sem0)
        cp_s = pltpu.async_copy(src_hbm.at[pl.ds(ebase, half)],
                                src_half.at[pl.ds(0, half)], sem1)

        zero16 = jnp.zeros((LANES,), jnp.float32)
        ones16 = jnp.full((LANES,), 1.0, jnp.float32)

        @pl.loop(0, Np // LANES)
        def _(i):
            deg_loc[pl.ds(i * LANES, LANES)] = zero16

        cp_z.wait()
        cp_s.wait()
        pltpu.sync_copy(dst_hbm.at[pl.ds(ebase, epw)], dst_all)
        plsc.subcore_barrier()

        rows = (rows0, rows1)
        sems = (sem0, sem1)
        ssems = (ssem0, ssem1)

        def start(b, i, coff):
            gi = src_half.at[pl.ds((i - coff) * CHUNK, CHUNK)]
            pltpu.async_copy(x_hbm.at[gi], rows[b], sems[b])

        def sref(b, i):
            return rows[b], acc_sh.at[dst_all.at[pl.ds(i * CHUNK, CHUNK)]]

        def work(b, i, coff):
            gi = src_half.at[pl.ds((i - coff) * CHUNK, CHUNK)]
            pltpu.make_async_copy(x_hbm.at[gi], rows[b], sems[b]).wait()
            src_r, dst_r = sref(b, i)
            pltpu.async_copy(src_r, dst_r, ssems[b], add=True)
            for k in range(CHUNK // LANES):
                idx = dst_all[pl.ds(i * CHUNK + k * LANES, LANES)]
                plsc.addupdate_scatter(deg_loc, [idx], ones16)

        def sdrain(b, i):
            src_r, dst_r = sref(b, i)
            pltpu.make_async_copy(src_r, dst_r, ssems[b]).wait()

        start(0, 0, 0)
        start(1, 1, 0)

        @pl.loop(0, na - 2, step=2)
        def _(i):
            work(0, i, 0)
            work(1, i + 1, 0)
            sdrain(0, i)
            start(0, i + 2, 0)
            sdrain(1, i + 1)
            start(1, i + 3, 0)

        work(0, na - 2, 0)
        work(1, na - 1, 0)
        sdrain(0, na - 2)
        sdrain(1, na - 1)

        pltpu.sync_copy(src_hbm.at[pl.ds(ebase + half, nb * CHUNK)],
                        src_half.at[pl.ds(0, nb * CHUNK)])
        start(0, na, na)
        start(1, na + 1, na)

        @pl.loop(na, nchunk - 3, step=2)
        def _(i):
            work(0, i, na)
            work(1, i + 1, na)
            sdrain(0, i)
            start(0, i + 2, na)
            sdrain(1, i + 1)
            start(1, i + 3, na)

        work(0, nchunk - 3, na)
        sdrain(0, nchunk - 3)
        start(0, nchunk - 1, na)
        work(1, nchunk - 2, na)
        work(0, nchunk - 1, na)
        sdrain(1, nchunk - 2)
        sdrain(0, nchunk - 1)

        plsc.subcore_barrier()
        pltpu.sync_copy(acc_sh.at[pl.ds(base_r, rps)],
                        agg_hbm.at[pl.ds(c * Np + base_r, rps)])
        pltpu.sync_copy(deg_loc, deg_hbm.at[pl.ds(w * Np, Np)])

    return agg_kernel(x, src, dst, zrow)


def _combine(x, agg_p, deg_p, W_self, W_neigh, b2):
    N, D = x.shape
    C = W_self.shape[1]
    NW = NC * NS
    BLK = 1280

    def body(x_ref, a_ref, d_ref, ws_ref, wn_ref, b_ref, o_ref):
        agg = a_ref[0] + a_ref[1]
        deg = jnp.sum(d_ref[...], axis=0)
        inv = 1.0 / jnp.maximum(deg, 1.0)
        hs = jnp.dot(x_ref[...], ws_ref[...],
                     preferred_element_type=jnp.float32)
        hn = jnp.dot(agg, wn_ref[...], preferred_element_type=jnp.float32)
        o_ref[...] = jnp.maximum(hs + hn * inv[:, None] + b_ref[...], 0.0)

    return pl.pallas_call(
        body,
        grid=(pl.cdiv(N, BLK),),
        in_specs=[
            pl.BlockSpec((BLK, D), lambda i: (i, 0)),
            pl.BlockSpec((NC, BLK, D), lambda i: (0, i, 0)),
            pl.BlockSpec((NW, BLK), lambda i: (0, i)),
            pl.BlockSpec((D, C), lambda i: (0, 0)),
            pl.BlockSpec((D, C), lambda i: (0, 0)),
            pl.BlockSpec((1, C), lambda i: (0, 0)),
        ],
        out_specs=pl.BlockSpec((BLK, C), lambda i: (i, 0)),
        out_shape=jax.ShapeDtypeStruct((N, C), jnp.float32),
    )(x, agg_p, deg_p, W_self, W_neigh, b2)


def kernel(x, edge_index, W_self, W_neigh, b):
    N, D = x.shape
    src = edge_index[0]
    dst = edge_index[1]
    NW = NC * NS
    rps = -(-N // (NS * ZROWS)) * ZROWS
    Np = NS * rps
    zrow = jnp.zeros((NS * rps, D), jnp.float32)
    agg_f, deg_f = _sc_aggregate(x, src, dst, zrow)
    agg_p = agg_f.reshape(NC, Np, D)
    deg_p = deg_f.reshape(NW, Np)
    return _combine(x, agg_p, deg_p, W_self, W_neigh, b.reshape(1, -1))


# speedup vs baseline: 1.1930x; 1.1930x over previous
import dataclasses
import functools

import jax
import jax.numpy as jnp
from jax import lax
from jax.experimental import pallas as pl
from jax.experimental.pallas import tpu as pltpu
from jax.experimental.pallas import tpu_sc as plsc

NC = 2
NS = 16
LANES = 16
CHUNK = 80
ZROWS = 128


def _sc_aggregate(x, src, dst, zrow):
    N, D = x.shape
    E = src.shape[0]
    NW = NC * NS
    assert E % (NW * CHUNK) == 0, (E, NW, CHUNK)
    epw = E // NW
    nchunk = epw // CHUNK
    na = nchunk // 2 + (nchunk // 2) % 2
    nb = nchunk - na
    assert na % 2 == 0 and nb % 2 == 1
    half = na * CHUNK
    bufsz = max(na, nb) * CHUNK
    rps = -(-N // (NS * ZROWS)) * ZROWS
    Np = NS * rps

    mesh = plsc.VectorSubcoreMesh(core_axis_name="c", subcore_axis_name="s")
    cp = pltpu.CompilerParams()
    if "needs_layout_passes" in pltpu.CompilerParams.__dataclass_fields__:
        cp = dataclasses.replace(cp, needs_layout_passes=False)

    @functools.partial(
        pl.kernel,
        out_type=[
            jax.ShapeDtypeStruct((NC * Np, D), jnp.float32),
            jax.ShapeDtypeStruct((NW * Np,), jnp.float32),
        ],
        mesh=mesh,
        compiler_params=cp,
        scratch_types=[
            pltpu.VMEM_SHARED((Np, D), jnp.float32),
            pltpu.VMEM((bufsz,), jnp.int32),
            pltpu.VMEM((epw,), jnp.int32),
            pltpu.VMEM((CHUNK, D), jnp.float32),
            pltpu.VMEM((CHUNK, D), jnp.float32),
            pltpu.VMEM((Np,), jnp.float32),
            pltpu.SemaphoreType.DMA,
            pltpu.SemaphoreType.DMA,
        ],
    )
    def agg_kernel(x_hbm, src_hbm, dst_hbm, zrow_hbm, agg_hbm, deg_hbm,
                   acc_sh, src_half, dst_all, rows0, rows1, deg_loc,
                   sem0, sem1):
        c = lax.axis_index("c")
        s = lax.axis_index("s")
        w = c * NS + s
        base_r = s * rps
        ebase = w * epw
        cp_z = pltpu.async_copy(zrow_hbm.at[pl.ds(base_r, rps)],
                                acc_sh.at[pl.ds(base_r, rps)], sem0)
        cp_s = pltpu.async_copy(src_hbm.at[pl.ds(ebase, half)],
                                src_half.at[pl.ds(0, half)], sem1)

        zero16 = jnp.zeros((LANES,), jnp.float32)
        ones16 = jnp.full((LANES,), 1.0, jnp.float32)

        @pl.loop(0, Np // LANES)
        def _(i):
            deg_loc[pl.ds(i * LANES, LANES)] = zero16

        cp_z.wait()
        cp_s.wait()
        pltpu.sync_copy(dst_hbm.at[pl.ds(ebase, epw)], dst_all)
        plsc.subcore_barrier()

        rows = (rows0, rows1)
        sems = (sem0, sem1)

        def start(b, i, coff):
            gi = src_half.at[pl.ds((i - coff) * CHUNK, CHUNK)]
            pltpu.async_copy(x_hbm.at[gi], rows[b], sems[b])

        def finish(b, i, coff):
            gi = src_half.at[pl.ds((i - coff) * CHUNK, CHUNK)]
            pltpu.make_async_copy(x_hbm.at[gi], rows[b], sems[b]).wait()
            pltpu.sync_copy(rows[b],
                            acc_sh.at[dst_all.at[pl.ds(i * CHUNK, CHUNK)]],
                            add=True)
            for k in range(CHUNK // LANES):
                idx = dst_all[pl.ds(i * CHUNK + k * LANES, LANES)]
                plsc.addupdate_scatter(deg_loc, [idx], ones16)

        start(0, 0, 0)

        @pl.loop(0, na - 2, step=2)
        def _(i):
            start(1, i + 1, 0)
            finish(0, i, 0)
            start(0, i + 2, 0)
            finish(1, i + 1, 0)

        start(1, na - 1, 0)
        finish(0, na - 2, 0)
        finish(1, na - 1, 0)

        pltpu.sync_copy(src_hbm.at[pl.ds(ebase + half, nb * CHUNK)],
                        src_half.at[pl.ds(0, nb * CHUNK)])
        start(0, na, na)

        @pl.loop(na, nchunk - 1, step=2)
        def _(i):
            start(1, i + 1, na)
            finish(0, i, na)
            start(0, i + 2, na)
            finish(1, i + 1, na)

        finish(0, nchunk - 1, na)

        plsc.subcore_barrier()
        pltpu.sync_copy(acc_sh.at[pl.ds(base_r, rps)],
                        agg_hbm.at[pl.ds(c * Np + base_r, rps)])
        pltpu.sync_copy(deg_loc, deg_hbm.at[pl.ds(w * Np, Np)])

    return agg_kernel(x, src, dst, zrow)


def _combine(x, agg_p, deg_p, W_self, W_neigh, b2):
    N, D = x.shape
    C = W_self.shape[1]
    NW = NC * NS
    BLK = 1280

    def body(x_ref, a_ref, d_ref, ws_ref, wn_ref, b_ref, o_ref):
        agg = a_ref[0] + a_ref[1]
        deg = jnp.sum(d_ref[...], axis=0)
        inv = 1.0 / jnp.maximum(deg, 1.0)
        hs = jnp.dot(x_ref[...], ws_ref[...],
                     preferred_element_type=jnp.float32)
        hn = jnp.dot(agg, wn_ref[...], preferred_element_type=jnp.float32)
        o_ref[...] = jnp.maximum(hs + hn * inv[:, None] + b_ref[...], 0.0)

    return pl.pallas_call(
        body,
        grid=(pl.cdiv(N, BLK),),
        in_specs=[
            pl.BlockSpec((BLK, D), lambda i: (i, 0)),
            pl.BlockSpec((NC, BLK, D), lambda i: (0, i, 0)),
            pl.BlockSpec((NW, BLK), lambda i: (0, i)),
            pl.BlockSpec((D, C), lambda i: (0, 0)),
            pl.BlockSpec((D, C), lambda i: (0, 0)),
            pl.BlockSpec((1, C), lambda i: (0, 0)),
        ],
        out_specs=pl.BlockSpec((BLK, C), lambda i: (i, 0)),
        out_shape=jax.ShapeDtypeStruct((N, C), jnp.float32),
    )(x, agg_p, deg_p, W_self, W_neigh, b2)


def kernel(x, edge_index, W_self, W_neigh, b):
    N, D = x.shape
    src = edge_index[0]
    dst = edge_index[1]
    NW = NC * NS
    rps = -(-N // (NS * ZROWS)) * ZROWS
    Np = NS * rps
    zrow = jnp.zeros((NS * rps, D), jnp.float32)
    agg_f, deg_f = _sc_aggregate(x, src, dst, zrow)
    agg_p = agg_f.reshape(NC, Np, D)
    deg_p = deg_f.reshape(NW, Np)
    return _combine(x, agg_p, deg_p, W_self, W_neigh, b.reshape(1, -1))
